# baseline (speedup 1.0000x reference)
import functools

import jax
import jax.numpy as jnp
from jax import lax
from jax.experimental import pallas as pl
from jax.experimental.pallas import tpu as pltpu
from jax.experimental.pallas import tpu_sc as plsc

_IGNORE_INDEX = -100
_MAX_ITER = 3
_LANES = 16
_NUM_CORES = 2
_NUM_SUBCORES = 16


def _argmax_body(x_ref, out_ref):
    x = x_ref[...]
    pred = jnp.argmax(x, axis=-1, keepdims=True)
    out_ref[...] = pred.astype(jnp.int32)


def _sc_assign_body(pred_hbm, lab_hbm, valid_hbm, mask_hbm, full_hbm,
                    depth_hbm, la_hbm, full_out_hbm,
                    pred_v, lab_v, valid_v, mask_v, full_v,
                    depth_v, la_v, prop_v, pos_v, out_v, *, B, S):
    cid = lax.axis_index("c")
    sid = lax.axis_index("s")

    @pl.when((cid == 0) & (sid < B))
    def _():
        row = sid
        pltpu.sync_copy(pred_hbm.at[row], pred_v)
        pltpu.sync_copy(lab_hbm.at[row], lab_v)
        pltpu.sync_copy(valid_hbm.at[row], valid_v)
        pltpu.sync_copy(mask_hbm.at[row], mask_v)
        pltpu.sync_copy(full_hbm.at[row], full_v)
        pltpu.sync_copy(depth_hbm, depth_v)

        d = depth_v[pl.ds(0, _LANES)]
        n_chunks = S // _LANES

        def body1(i, carry):
            sl = pl.ds(i * _LANES, _LANES)
            pred = pred_v[sl]
            lab = lab_v[sl]
            valid = valid_v[sl]
            mv = mask_v[sl]
            s_glob = lax.iota(jnp.int32, _LANES) + i * _LANES
            cont = (pred != lab) & (s_glob < S - 1) & (lab != _IGNORE_INDEX)
            la = jnp.where(cont, d + 1, d)
            la = jnp.minimum(la, _MAX_ITER)
            la = jnp.where(valid == 1, la, _IGNORE_INDEX)
            la_v[sl] = la
            prop_v[sl] = jnp.where(la == _IGNORE_INDEX, 0, la)
            cs = plsc.cumsum(mv) + carry
            pos_v[sl] = jnp.clip(cs - 1, 0, S - 1)
            return jnp.max(cs)

        lax.fori_loop(0, n_chunks, body1, jnp.int32(0))

        def body2(i, carry):
            sl = pl.ds(i * _LANES, _LANES)
            pos = pos_v[sl]
            g = plsc.load_gather(prop_v, [pos])
            mv = mask_v[sl]
            fv = full_v[sl]
            out_v[sl] = jnp.maximum(fv, jnp.where(mv != 0, g, 0))
            return carry

        lax.fori_loop(0, n_chunks, body2, jnp.int32(0))

        pltpu.sync_copy(la_v, la_hbm.at[row])
        pltpu.sync_copy(out_v, full_out_hbm.at[row])


def kernel(active_logits, active_labels_shifted, iter_depth,
           current_iter_mask, active_valid_mask, full_labels):
    B, S, V = active_logits.shape
    BS = 2048
    n_sblk = S // BS

    lab = active_labels_shifted.astype(jnp.int32)
    valid = active_valid_mask.astype(jnp.int32)
    maskv = current_iter_mask.astype(jnp.int32)
    full = full_labels.astype(jnp.int32)
    depth = jnp.full((_LANES,), iter_depth, dtype=jnp.int32)

    mesh = plsc.VectorSubcoreMesh(
        core_axis_name="c", subcore_axis_name="s",
        num_cores=_NUM_CORES, num_subcores=_NUM_SUBCORES)
    row_i32 = functools.partial(pltpu.VMEM, (S,), jnp.int32)

    HB = B // 2
    sc_call = pl.kernel(
        functools.partial(_sc_assign_body, B=HB, S=S),
        out_type=[jax.ShapeDtypeStruct((HB, S), jnp.int32),
                  jax.ShapeDtypeStruct((HB, S), jnp.int32)],
        mesh=mesh,
        scratch_types=[row_i32(), row_i32(), row_i32(), row_i32(), row_i32(),
                       pltpu.VMEM((_LANES,), jnp.int32),
                       row_i32(), row_i32(), row_i32(), row_i32()],
        compiler_params=pltpu.CompilerParams(needs_layout_passes=False),
    )

    def tc_argmax(x):
        nb = x.shape[0]
        pred = pl.pallas_call(
            _argmax_body,
            grid=(nb, n_sblk),
            in_specs=[pl.BlockSpec((1, BS, V), lambda b, s: (b, s, 0))],
            out_specs=pl.BlockSpec((1, BS, 1), lambda b, s: (b, s, 0)),
            out_shape=jax.ShapeDtypeStruct((nb, BS, 1), jnp.int32),
            compiler_params=pltpu.CompilerParams(
                vmem_limit_bytes=100 * 1024 * 1024),
        )(x)
        return pred.reshape(nb, S)

    outs = []
    for h in range(2):
        rows = slice(h * HB, (h + 1) * HB)
        pred_h = tc_argmax(active_logits[rows])
        outs.append(sc_call(pred_h, lab[rows], valid[rows], maskv[rows],
                            full[rows], depth))
    la = jnp.concatenate([outs[0][0], outs[1][0]], axis=0)
    full_new = jnp.concatenate([outs[0][1], outs[1][1]], axis=0)
    return la, full_new


# speedup vs baseline: 2.2166x; 2.2166x over previous
import functools

import jax
import jax.numpy as jnp
from jax import lax
from jax.experimental import pallas as pl
from jax.experimental.pallas import tpu as pltpu
from jax.experimental.pallas import tpu_sc as plsc

_IGNORE_INDEX = -100
_MAX_ITER = 3
_LANES = 16
_NUM_CORES = 2
_NUM_SUBCORES = 16


def _argmax_body(x_ref, out_ref):
    x = x_ref[...]
    pred = jnp.argmax(x, axis=-1, keepdims=True)
    out_ref[...] = pred.astype(jnp.int32)


def _sc_assign_body(pred_hbm, lab_hbm, valid_hbm, mask_hbm, full_hbm,
                    depth_hbm, la_hbm, full_out_hbm,
                    pred_v, lab_v, valid_v, mask_v, full_v,
                    depth_v, la_v, prop_v, pos_v, out_v, *, B, S):
    cid = lax.axis_index("c")
    sid = lax.axis_index("s")

    @pl.when((cid == 0) & (sid < B))
    def _():
        row = sid
        pltpu.sync_copy(pred_hbm.at[row], pred_v)
        pltpu.sync_copy(lab_hbm.at[row], lab_v)
        pltpu.sync_copy(valid_hbm.at[row], valid_v)
        pltpu.sync_copy(mask_hbm.at[row], mask_v)
        pltpu.sync_copy(full_hbm.at[row], full_v)
        pltpu.sync_copy(depth_hbm, depth_v)

        d = depth_v[pl.ds(0, _LANES)]
        n_chunks = S // _LANES

        def body1(i, carry):
            sl = pl.ds(i * _LANES, _LANES)
            pred = pred_v[sl]
            lab = lab_v[sl]
            valid = valid_v[sl]
            mv = mask_v[sl]
            s_glob = lax.iota(jnp.int32, _LANES) + i * _LANES
            cont = (pred != lab) & (s_glob < S - 1) & (lab != _IGNORE_INDEX)
            la = jnp.where(cont, d + 1, d)
            la = jnp.minimum(la, _MAX_ITER)
            la = jnp.where(valid == 1, la, _IGNORE_INDEX)
            la_v[sl] = la
            prop_v[sl] = jnp.where(la == _IGNORE_INDEX, 0, la)
            cs = plsc.cumsum(mv) + carry
            pos_v[sl] = jnp.clip(cs - 1, 0, S - 1)
            return jnp.max(cs)

        lax.fori_loop(0, n_chunks, body1, jnp.int32(0))

        def body2(i, carry):
            sl = pl.ds(i * _LANES, _LANES)
            pos = pos_v[sl]
            g = plsc.load_gather(prop_v, [pos])
            mv = mask_v[sl]
            fv = full_v[sl]
            out_v[sl] = jnp.maximum(fv, jnp.where(mv != 0, g, 0))
            return carry

        lax.fori_loop(0, n_chunks, body2, jnp.int32(0))

        pltpu.sync_copy(la_v, la_hbm.at[row])
        pltpu.sync_copy(out_v, full_out_hbm.at[row])


def kernel(active_logits, active_labels_shifted, iter_depth,
           current_iter_mask, active_valid_mask, full_labels):
    B, S, V = active_logits.shape
    BS = 2048
    n_sblk = S // BS

    lab = active_labels_shifted.astype(jnp.int32)
    valid = active_valid_mask.astype(jnp.int32)
    maskv = current_iter_mask.astype(jnp.int32)
    full = full_labels.astype(jnp.int32)
    depth = jnp.full((_LANES,), iter_depth, dtype=jnp.int32)

    mesh = plsc.VectorSubcoreMesh(
        core_axis_name="c", subcore_axis_name="s",
        num_cores=_NUM_CORES, num_subcores=_NUM_SUBCORES)
    row_i32 = functools.partial(pltpu.VMEM, (S,), jnp.int32)

    HB = B // 2
    sc_call = pl.kernel(
        functools.partial(_sc_assign_body, B=HB, S=S),
        out_type=[jax.ShapeDtypeStruct((HB, S), jnp.int32),
                  jax.ShapeDtypeStruct((HB, S), jnp.int32)],
        mesh=mesh,
        scratch_types=[row_i32(), row_i32(), row_i32(), row_i32(), row_i32(),
                       pltpu.VMEM((_LANES,), jnp.int32),
                       row_i32(), row_i32(), row_i32(), row_i32()],
        compiler_params=pltpu.CompilerParams(needs_layout_passes=False),
    )

    def tc_argmax(row0):
        pred = pl.pallas_call(
            _argmax_body,
            grid=(HB, n_sblk),
            in_specs=[pl.BlockSpec((1, BS, V),
                                   lambda b, s: (b + row0, s, 0))],
            out_specs=pl.BlockSpec((1, BS, 1), lambda b, s: (b, s, 0)),
            out_shape=jax.ShapeDtypeStruct((HB, BS, 1), jnp.int32),
            compiler_params=pltpu.CompilerParams(
                vmem_limit_bytes=100 * 1024 * 1024),
        )(active_logits)
        return pred.reshape(HB, S)

    outs = []
    for h in range(2):
        rows = slice(h * HB, (h + 1) * HB)
        pred_h = tc_argmax(h * HB)
        outs.append(sc_call(pred_h, lab[rows], valid[rows], maskv[rows],
                            full[rows], depth))
    la = jnp.concatenate([outs[0][0], outs[1][0]], axis=0)
    full_new = jnp.concatenate([outs[0][1], outs[1][1]], axis=0)
    return la, full_new


# speedup vs baseline: 2.4728x; 1.1156x over previous
import functools

import jax
import jax.numpy as jnp
from jax import lax
from jax.experimental import pallas as pl
from jax.experimental.pallas import tpu as pltpu
from jax.experimental.pallas import tpu_sc as plsc

_IGNORE_INDEX = -100
_MAX_ITER = 3
_LANES = 16
_NUM_CORES = 2
_NUM_SUBCORES = 16


def _argmax_body(x_ref, out_ref):
    x = x_ref[...]
    pred = jnp.argmax(x, axis=-1, keepdims=True)
    out_ref[...] = pred.astype(jnp.int32)


def _sc_assign_body(pred_hbm, lab_hbm, valid_hbm, mask_hbm, full_hbm,
                    depth_hbm, la_hbm, full_out_hbm,
                    pred_v, lab_v, valid_v, mask_v, full_v,
                    depth_v, la_v, prop_v, out_v, *, B, S):
    cid = lax.axis_index("c")
    sid = lax.axis_index("s")

    @pl.when((cid == 0) & (sid < B))
    def _():
        row = sid
        pltpu.sync_copy(pred_hbm.at[row], pred_v)
        pltpu.sync_copy(lab_hbm.at[row], lab_v)
        pltpu.sync_copy(valid_hbm.at[row], valid_v)
        pltpu.sync_copy(mask_hbm.at[row], mask_v)
        pltpu.sync_copy(full_hbm.at[row], full_v)
        pltpu.sync_copy(depth_hbm, depth_v)

        d = depth_v[pl.ds(0, _LANES)]
        n_chunks = S // _LANES

        def body1(i, carry):
            sl = pl.ds(i * _LANES, _LANES)
            pred = pred_v[sl]
            lab = lab_v[sl]
            valid = valid_v[sl]
            mv = mask_v[sl]
            s_glob = lax.iota(jnp.int32, _LANES) + i * _LANES
            cont = (pred != lab) & (s_glob < S - 1) & (lab != _IGNORE_INDEX)
            la = jnp.where(cont, d + 1, d)
            la = jnp.minimum(la, _MAX_ITER)
            la = jnp.where(valid == 1, la, _IGNORE_INDEX)
            la_v[sl] = la
            prop_v[sl] = jnp.where(la == _IGNORE_INDEX, 0, la)
            cs = plsc.cumsum(mv) + carry
            pos = jnp.clip(cs - 1, 0, S - 1)
            g = plsc.load_gather(prop_v, [pos])
            fv = full_v[sl]
            out_v[sl] = jnp.maximum(fv, jnp.where(mv != 0, g, 0))
            return jnp.max(cs)

        lax.fori_loop(0, n_chunks, body1, jnp.int32(0))

        pltpu.sync_copy(la_v, la_hbm.at[row])
        pltpu.sync_copy(out_v, full_out_hbm.at[row])


def kernel(active_logits, active_labels_shifted, iter_depth,
           current_iter_mask, active_valid_mask, full_labels):
    B, S, V = active_logits.shape
    BS = 2048
    n_sblk = S // BS

    lab = active_labels_shifted.astype(jnp.int32)
    valid = active_valid_mask.astype(jnp.int32)
    maskv = current_iter_mask.astype(jnp.int32)
    full = full_labels.astype(jnp.int32)
    depth = jnp.full((_LANES,), iter_depth, dtype=jnp.int32)

    mesh = plsc.VectorSubcoreMesh(
        core_axis_name="c", subcore_axis_name="s",
        num_cores=_NUM_CORES, num_subcores=_NUM_SUBCORES)
    row_i32 = functools.partial(pltpu.VMEM, (S,), jnp.int32)

    HB = B
    sc_call = pl.kernel(
        functools.partial(_sc_assign_body, B=HB, S=S),
        out_type=[jax.ShapeDtypeStruct((HB, S), jnp.int32),
                  jax.ShapeDtypeStruct((HB, S), jnp.int32)],
        mesh=mesh,
        scratch_types=[row_i32(), row_i32(), row_i32(), row_i32(), row_i32(),
                       pltpu.VMEM((_LANES,), jnp.int32),
                       row_i32(), row_i32(), row_i32()],
        compiler_params=pltpu.CompilerParams(needs_layout_passes=False),
    )

    def tc_argmax(row0):
        pred = pl.pallas_call(
            _argmax_body,
            grid=(HB, n_sblk),
            in_specs=[pl.BlockSpec((1, BS, V),
                                   lambda b, s: (b + row0, s, 0))],
            out_specs=pl.BlockSpec((1, BS, 1), lambda b, s: (b, s, 0)),
            out_shape=jax.ShapeDtypeStruct((HB, BS, 1), jnp.int32),
            compiler_params=pltpu.CompilerParams(
                vmem_limit_bytes=100 * 1024 * 1024),
        )(active_logits)
        return pred.reshape(HB, S)

    pred = tc_argmax(0)
    la, full_new = sc_call(pred, lab, valid, maskv, full, depth)
    return la, full_new


# speedup vs baseline: 2.5042x; 1.0127x over previous
import functools

import jax
import jax.numpy as jnp
from jax import lax
from jax.experimental import pallas as pl
from jax.experimental.pallas import tpu as pltpu
from jax.experimental.pallas import tpu_sc as plsc

_IGNORE_INDEX = -100
_MAX_ITER = 3
_LANES = 16
_NUM_CORES = 2
_NUM_SUBCORES = 16


def _argmax_body(x_ref, out_ref):
    x = x_ref[...]
    pred = jnp.argmax(x, axis=-1, keepdims=True)
    out_ref[...] = pred.astype(jnp.int32)


def _sc_assign_body(pred_hbm, lab_hbm, valid_hbm, mask_hbm, full_hbm,
                    depth_hbm, la_hbm, full_out_hbm,
                    pred_v, lab_v, valid_v, mask_v, full_v,
                    depth_v, la_v, prop_v, out_v, *, B, S):
    cid = lax.axis_index("c")
    sid = lax.axis_index("s")

    @pl.when((cid == 0) & (sid < B))
    def _():
        row = sid
        pltpu.sync_copy(pred_hbm.at[row], pred_v)
        pltpu.sync_copy(lab_hbm.at[row], lab_v)
        pltpu.sync_copy(valid_hbm.at[row], valid_v)
        pltpu.sync_copy(mask_hbm.at[row], mask_v)
        pltpu.sync_copy(full_hbm.at[row], full_v)
        pltpu.sync_copy(depth_hbm, depth_v)

        d = depth_v[pl.ds(0, _LANES)]
        n_chunks = S // _LANES

        def body1(i, carry):
            sl = pl.ds(i * _LANES, _LANES)
            pred = pred_v[sl]
            lab = lab_v[sl]
            valid = valid_v[sl]
            mv = mask_v[sl]
            s_glob = lax.iota(jnp.int32, _LANES) + i * _LANES
            cont = (pred != lab) & (s_glob < S - 1) & (lab != _IGNORE_INDEX)
            la = jnp.where(cont, d + 1, d)
            la = jnp.minimum(la, _MAX_ITER)
            la = jnp.where(valid == 1, la, _IGNORE_INDEX)
            la_v[sl] = la
            prop_v[sl] = jnp.where(la == _IGNORE_INDEX, 0, la)
            cs = plsc.cumsum(mv) + carry
            pos = jnp.clip(cs - 1, 0, S - 1)
            g = plsc.load_gather(prop_v, [pos])
            fv = full_v[sl]
            out_v[sl] = jnp.maximum(fv, jnp.where(mv != 0, g, 0))
            return jnp.max(cs)

        lax.fori_loop(0, n_chunks, body1, jnp.int32(0))

        pltpu.sync_copy(la_v, la_hbm.at[row])
        pltpu.sync_copy(out_v, full_out_hbm.at[row])


def kernel(active_logits, active_labels_shifted, iter_depth,
           current_iter_mask, active_valid_mask, full_labels):
    B, S, V = active_logits.shape
    BS = 2048
    n_sblk = S // BS

    lab = active_labels_shifted.astype(jnp.int32)
    valid = active_valid_mask.astype(jnp.int32)
    maskv = current_iter_mask.astype(jnp.int32)
    full = full_labels.astype(jnp.int32)
    depth = jnp.full((_LANES,), iter_depth, dtype=jnp.int32)

    mesh = plsc.VectorSubcoreMesh(
        core_axis_name="c", subcore_axis_name="s",
        num_cores=_NUM_CORES, num_subcores=_NUM_SUBCORES)
    row_i32 = functools.partial(pltpu.VMEM, (S,), jnp.int32)

    HB = B
    sc_call = pl.kernel(
        functools.partial(_sc_assign_body, B=HB, S=S),
        out_type=[jax.ShapeDtypeStruct((HB, S), jnp.int32),
                  jax.ShapeDtypeStruct((HB, S), jnp.int32)],
        mesh=mesh,
        scratch_types=[row_i32(), row_i32(), row_i32(), row_i32(), row_i32(),
                       pltpu.VMEM((_LANES,), jnp.int32),
                       row_i32(), row_i32(), row_i32()],
        compiler_params=pltpu.CompilerParams(needs_layout_passes=False),
    )

    def tc_argmax(row0):
        pred = pl.pallas_call(
            _argmax_body,
            grid=(HB, n_sblk),
            in_specs=[pl.BlockSpec((1, BS, V),
                                   lambda b, s: (b + row0, s, 0))],
            out_specs=pl.BlockSpec((1, BS, 1), lambda b, s: (b, s, 0)),
            out_shape=jax.ShapeDtypeStruct((HB, BS, 1), jnp.int32),
            compiler_params=pltpu.CompilerParams(
                dimension_semantics=("arbitrary", "arbitrary"),
                vmem_limit_bytes=100 * 1024 * 1024),
        )(active_logits)
        return pred.reshape(HB, S)

    pred = tc_argmax(0)
    la, full_new = sc_call(pred, lab, valid, maskv, full, depth)
    return la, full_new


# speedup vs baseline: 2.5163x; 1.0048x over previous
import functools

import jax
import jax.numpy as jnp
from jax import lax
from jax.experimental import pallas as pl
from jax.experimental.pallas import tpu as pltpu
from jax.experimental.pallas import tpu_sc as plsc

_IGNORE_INDEX = -100
_MAX_ITER = 3
_LANES = 16
_NUM_CORES = 2
_NUM_SUBCORES = 16


def _argmax_body(x_ref, out_ref):
    x = x_ref[...]
    pred = jnp.argmax(x, axis=-1, keepdims=True)
    out_ref[...] = pred.astype(jnp.int32)


def _sc_pos_body(mask_hbm, pos_hbm, mask_v, pos_v, *, B, S):
    cid = lax.axis_index("c")
    sid = lax.axis_index("s")

    @pl.when((cid == 0) & (sid < B))
    def _():
        row = sid
        pltpu.sync_copy(mask_hbm.at[row], mask_v)
        n_chunks = S // _LANES

        def body(i, carry):
            sl = pl.ds(i * _LANES, _LANES)
            mv = mask_v[sl]
            cs = plsc.cumsum(mv) + carry
            pos_v[sl] = jnp.clip(cs - 1, 0, S - 1)
            cnt = plsc.all_reduce_population_count(mv != 0)
            return carry + cnt

        lax.fori_loop(0, n_chunks, body,
                      jnp.zeros((_LANES,), dtype=jnp.int32))
        pltpu.sync_copy(pos_v, pos_hbm.at[row])


def _sc_assign_body(pred_hbm, lab_hbm, valid_hbm, mask_hbm, full_hbm,
                    pos_hbm, depth_hbm, la_hbm, full_out_hbm,
                    pred_v, lab_v, valid_v, mask_v, full_v, pos_v,
                    depth_v, la_v, prop_v, out_v, *, B, S):
    cid = lax.axis_index("c")
    sid = lax.axis_index("s")

    @pl.when((cid == 0) & (sid < B))
    def _():
        row = sid
        pltpu.sync_copy(pred_hbm.at[row], pred_v)
        pltpu.sync_copy(lab_hbm.at[row], lab_v)
        pltpu.sync_copy(valid_hbm.at[row], valid_v)
        pltpu.sync_copy(mask_hbm.at[row], mask_v)
        pltpu.sync_copy(full_hbm.at[row], full_v)
        pltpu.sync_copy(pos_hbm.at[row], pos_v)
        pltpu.sync_copy(depth_hbm, depth_v)

        d = depth_v[pl.ds(0, _LANES)]
        n_chunks = S // _LANES

        def body1(i, carry):
            sl = pl.ds(i * _LANES, _LANES)
            pred = pred_v[sl]
            lab = lab_v[sl]
            valid = valid_v[sl]
            mv = mask_v[sl]
            s_glob = lax.iota(jnp.int32, _LANES) + i * _LANES
            cont = (pred != lab) & (s_glob < S - 1) & (lab != _IGNORE_INDEX)
            la = jnp.where(cont, d + 1, d)
            la = jnp.minimum(la, _MAX_ITER)
            la = jnp.where(valid == 1, la, _IGNORE_INDEX)
            la_v[sl] = la
            prop_v[sl] = jnp.where(la == _IGNORE_INDEX, 0, la)
            g = plsc.load_gather(prop_v, [pos_v[sl]])
            fv = full_v[sl]
            out_v[sl] = jnp.maximum(fv, jnp.where(mv != 0, g, 0))
            return carry

        lax.fori_loop(0, n_chunks, body1, jnp.int32(0))

        pltpu.sync_copy(la_v, la_hbm.at[row])
        pltpu.sync_copy(out_v, full_out_hbm.at[row])


def kernel(active_logits, active_labels_shifted, iter_depth,
           current_iter_mask, active_valid_mask, full_labels):
    B, S, V = active_logits.shape
    BS = 2048
    n_sblk = S // BS

    lab = active_labels_shifted.astype(jnp.int32)
    valid = active_valid_mask.astype(jnp.int32)
    maskv = current_iter_mask.astype(jnp.int32)
    full = full_labels.astype(jnp.int32)
    depth = jnp.full((_LANES,), iter_depth, dtype=jnp.int32)

    mesh = plsc.VectorSubcoreMesh(
        core_axis_name="c", subcore_axis_name="s",
        num_cores=_NUM_CORES, num_subcores=_NUM_SUBCORES)
    row_i32 = functools.partial(pltpu.VMEM, (S,), jnp.int32)

    HB = B
    sc_pos = pl.kernel(
        functools.partial(_sc_pos_body, B=B, S=S),
        out_type=jax.ShapeDtypeStruct((B, S), jnp.int32),
        mesh=mesh,
        scratch_types=[row_i32(), row_i32()],
        compiler_params=pltpu.CompilerParams(needs_layout_passes=False),
    )
    sc_call = pl.kernel(
        functools.partial(_sc_assign_body, B=HB, S=S),
        out_type=[jax.ShapeDtypeStruct((HB, S), jnp.int32),
                  jax.ShapeDtypeStruct((HB, S), jnp.int32)],
        mesh=mesh,
        scratch_types=[row_i32(), row_i32(), row_i32(), row_i32(), row_i32(),
                       row_i32(),
                       pltpu.VMEM((_LANES,), jnp.int32),
                       row_i32(), row_i32(), row_i32()],
        compiler_params=pltpu.CompilerParams(needs_layout_passes=False),
    )

    def tc_argmax(row0):
        pred = pl.pallas_call(
            _argmax_body,
            grid=(HB, n_sblk),
            in_specs=[pl.BlockSpec((1, BS, V),
                                   lambda b, s: (b + row0, s, 0))],
            out_specs=pl.BlockSpec((1, BS, 1), lambda b, s: (b, s, 0)),
            out_shape=jax.ShapeDtypeStruct((HB, BS, 1), jnp.int32),
            compiler_params=pltpu.CompilerParams(
                dimension_semantics=("arbitrary", "arbitrary"),
                vmem_limit_bytes=100 * 1024 * 1024),
        )(active_logits)
        return pred.reshape(HB, S)

    pos = sc_pos(maskv)
    pred = tc_argmax(0)
    la, full_new = sc_call(pred, lab, valid, maskv, full, pos, depth)
    return la, full_new
